# baseline (speedup 1.0000x reference)
import functools

import jax
import jax.numpy as jnp
from jax import lax
from jax.experimental import pallas as pl
from jax.experimental.pallas import tpu as pltpu
from jax.experimental.pallas import tpu_sc as plsc

NC = 2
NS = 16
CHUNK = 64
TRASH = 32
EBLK = 512


def _tc_y2_body(x_ref, w_ref, b_ref, out_ref):
    out_ref[...] = lax.dot_general(
        x_ref[...], w_ref[0],
        (((1,), (0,)), ((), ())),
        preferred_element_type=jnp.float32) + b_ref[0]


def _tc_edge_body(ea_ref, eam_ref, wi_ref, wd_ref, easel_ref):
    easel_ref[...] = (
        lax.dot_general(ea_ref[...], wi_ref[...],
                        (((1,), (0,)), ((), ())),
                        preferred_element_type=jnp.float32)
        + lax.dot_general(eam_ref[...], wd_ref[...],
                          (((1,), (0,)), ((), ())),
                          preferred_element_type=jnp.float32))


def _tc_node_body(fi_ref, fo_ref, wa_ref, wb_ref, b_ref, out_ref):
    acc = lax.dot_general(fi_ref[...], wa_ref[...],
                          (((1,), (0,)), ((), ())),
                          preferred_element_type=jnp.float32)
    acc += lax.dot_general(fo_ref[...], wb_ref[...],
                           (((1,), (0,)), ((), ())),
                           preferred_element_type=jnp.float32)
    out_ref[...] = jnp.maximum(acc + b_ref[...], 0.0)


def _sc_body(n_nodes, epad, d, y2_ref, ea_ref, row_ref, col_ref, out_ref,
             acc, rwb0, rwb1, clb0, clb1, igb0, igb1, irb0, irb1, eab0, eab1,
             yb0, yb1, sr0, sr1, sc0, sc1, se0, se1, sy0, sy1):
    c = lax.axis_index("c")
    s = lax.axis_index("s")
    nacc = n_nodes + TRASH
    zstride = (-(-nacc // NS) + 7) // 8 * 8
    zbase = jnp.minimum(s * zstride, nacc - zstride)
    wstride = (-(-n_nodes // NS) + 7) // 8 * 8
    wbase = jnp.minimum(s * wstride, n_nodes - wstride)
    edges_per_tile = epad // NS
    nsteps = edges_per_tile // CHUNK
    assert nsteps % 2 == 0
    nvec = d // 16

    rwb = (rwb0, rwb1)
    clb = (clb0, clb1)
    igb = (igb0, igb1)
    irb = (irb0, irb1)
    eab = (eab0, eab1)
    yb = (yb0, yb1)
    sr = (sr0, sr1)
    sc_ = (sc0, sc1)
    se = (se0, se1)
    sy = (sy0, sy1)

    ebase = s * edges_per_tile
    sdir = 2 * c - 1

    def issue_lin(t, p):
        pltpu.async_copy(row_ref.at[pl.ds(ebase + t * CHUNK, CHUNK)],
                         rwb[p], sr[p])
        pltpu.async_copy(col_ref.at[pl.ds(ebase + t * CHUNK, CHUNK)],
                         clb[p], sc_[p])
        pltpu.async_copy(ea_ref.at[pl.ds(ebase + t * CHUNK, CHUNK)],
                         eab[p], se[p])

    def gather_start(t, p):
        pltpu.make_async_copy(row_ref.at[pl.ds(ebase, CHUNK)], rwb[p],
                              sr[p]).wait()
        pltpu.make_async_copy(col_ref.at[pl.ds(ebase, CHUNK)], clb[p],
                              sc_[p]).wait()
        tro = t * CHUNK
        for k in range(CHUNK // 16):
            sl = pl.ds(k * 16, 16)
            r = rwb[p][sl]
            cc = clb[p][sl]
            m_out = cc > r
            igb[p][sl] = jnp.where(m_out, cc + n_nodes, cc)
            live = (cc - r) * sdir > 0
            trash = n_nodes + ((tro + k * 16 +
                                lax.iota(jnp.int32, 16)) & (TRASH - 1))
            irb[p][sl] = jnp.where(live, r, trash)
        pltpu.async_copy(y2_ref.at[igb[p]], yb[p], sy[p])

    def compute(p):
        pltpu.make_async_copy(ea_ref.at[pl.ds(ebase, CHUNK)], eab[p],
                              se[p]).wait()
        pltpu.make_async_copy(y2_ref.at[igb[p]], yb[p], sy[p]).wait()

        def crow(r, carry):
            for k in range(nvec):
                sl = pl.ds(k * 16, 16)
                eab[p][r, sl] = jnp.maximum(eab[p][r, sl] + yb[p][r, sl],
                                            0.0)
            return carry
        lax.fori_loop(0, CHUNK, crow, 0)

    def scat(p):
        pltpu.sync_copy(eab[p], acc.at[irb[p]], add=True)

    def zrow(r, carry):
        for k in range(nvec):
            eab0[r, pl.ds(k * 16, 16)] = jnp.zeros((16,), jnp.float32)
        return carry
    lax.fori_loop(0, CHUNK, zrow, 0)
    full, rem = divmod(zstride, CHUNK)
    for j in range(full):
        pltpu.sync_copy(eab0, acc.at[pl.ds(zbase + j * CHUNK, CHUNK)])
    if rem:
        pltpu.sync_copy(eab0.at[pl.ds(0, rem)],
                        acc.at[pl.ds(zbase + full * CHUNK, rem)])
    plsc.subcore_barrier()

    issue_lin(0, 0)
    gather_start(0, 0)

    def pair(tt, carry):
        t0 = 2 * tt
        issue_lin(t0 + 1, 1)
        compute(0)
        gather_start(t0 + 1, 1)
        scat(0)

        @pl.when(t0 + 2 < nsteps)
        def _():
            issue_lin(t0 + 2, 0)
        compute(1)

        @pl.when(t0 + 2 < nsteps)
        def _():
            gather_start(t0 + 2, 0)
        scat(1)
        return carry
    lax.fori_loop(0, nsteps // 2, pair, 0)

    plsc.subcore_barrier()
    pltpu.sync_copy(acc.at[pl.ds(wbase, wstride)],
                    out_ref.at[pl.ds(c * n_nodes + wbase, wstride)])


def kernel(x, edge_index, edge_attr, W_in, b_in, W_out, b_out, W_node,
           b_node):
    n, d = x.shape
    e = edge_index.shape[1]
    de = edge_attr.shape[1]
    gran = NS * CHUNK * 2
    assert gran % EBLK == 0 and e % EBLK == 0
    epad = ((e + gran - 1) // gran) * gran
    neblk = epad // EBLK
    nfull = e // EBLK

    rowp = jnp.pad(edge_index[0], (0, epad - e))
    colp = jnp.pad(edge_index[1], (0, epad - e))
    eam = edge_attr * (edge_index[1] > edge_index[0]
                       ).astype(jnp.float32)[:, None]

    wx = jnp.stack([W_in[:d], W_out[:d]])
    b2 = jnp.stack([b_in, b_out]).reshape(2, 1, d)
    nblk_y = 10
    rows_y = n // nblk_y
    y2 = pl.pallas_call(
        _tc_y2_body,
        grid=(2, nblk_y),
        in_specs=[
            pl.BlockSpec((rows_y, d), lambda dd, i: (i, 0)),
            pl.BlockSpec((1, d, d), lambda dd, i: (dd, 0, 0)),
            pl.BlockSpec((1, 1, d), lambda dd, i: (dd, 0, 0)),
        ],
        out_specs=pl.BlockSpec((rows_y, d),
                               lambda dd, i: (dd * nblk_y + i, 0)),
        out_shape=jax.ShapeDtypeStruct((2 * n, d), jnp.float32),
    )(x, wx, b2)

    ea_sel = pl.pallas_call(
        _tc_edge_body,
        grid=(neblk,),
        in_specs=[
            pl.BlockSpec((EBLK, de),
                         lambda i: (jnp.minimum(i, nfull - 1), 0)),
            pl.BlockSpec((EBLK, de),
                         lambda i: (jnp.minimum(i, nfull - 1), 0)),
            pl.BlockSpec((de, d), lambda i: (0, 0)),
            pl.BlockSpec((de, d), lambda i: (0, 0)),
        ],
        out_specs=pl.BlockSpec((EBLK, d), lambda i: (i, 0)),
        out_shape=jax.ShapeDtypeStruct((epad, d), jnp.float32),
    )(edge_attr, eam, W_in[d:], W_out[d:] - W_in[d:])

    mesh = plsc.VectorSubcoreMesh(core_axis_name="c", subcore_axis_name="s",
                                  num_cores=NC, num_subcores=NS)
    flow = pl.kernel(
        functools.partial(_sc_body, n, epad, d),
        out_type=jax.ShapeDtypeStruct((2 * n, d), jnp.float32),
        mesh=mesh,
        scratch_types=(
            [pltpu.MemorySpace.VMEM_SHARED((n + TRASH, d), jnp.float32)]
            + [pltpu.VMEM((CHUNK,), jnp.int32)] * 8
            + [pltpu.VMEM((CHUNK, d), jnp.float32)] * 4
            + [pltpu.SemaphoreType.DMA] * 8
        ),
    )(y2, ea_sel, rowp, colp)

    nblk = 10
    rows_n = n // nblk
    out = pl.pallas_call(
        _tc_node_body,
        grid=(nblk,),
        in_specs=[
            pl.BlockSpec((rows_n, d), lambda i: (i, 0)),
            pl.BlockSpec((rows_n, d), lambda i: (nblk + i, 0)),
            pl.BlockSpec((d, d), lambda i: (0, 0)),
            pl.BlockSpec((d, d), lambda i: (1, 0)),
            pl.BlockSpec((1, d), lambda i: (0, 0)),
        ],
        out_specs=pl.BlockSpec((rows_n, d), lambda i: (i, 0)),
        out_shape=jax.ShapeDtypeStruct((n, d), jnp.float32),
    )(flow, flow, W_node, W_node, b_node.reshape(1, d))
    return out


# speedup vs baseline: 1.0682x; 1.0682x over previous
import functools

import jax
import jax.numpy as jnp
from jax import lax
from jax.experimental import pallas as pl
from jax.experimental.pallas import tpu as pltpu
from jax.experimental.pallas import tpu_sc as plsc

NC = 2
NS = 16
CHUNK = 64
TRASH = 32
EBLK = 512


def _tc_y2_body(x_ref, w_ref, b_ref, out_ref):
    out_ref[...] = lax.dot_general(
        x_ref[...], w_ref[0],
        (((1,), (0,)), ((), ())),
        preferred_element_type=jnp.float32) + b_ref[0]


def _tc_edge_body(ea_ref, eam_ref, wi_ref, wd_ref, easel_ref):
    easel_ref[...] = (
        lax.dot_general(ea_ref[...], wi_ref[...],
                        (((1,), (0,)), ((), ())),
                        preferred_element_type=jnp.float32)
        + lax.dot_general(eam_ref[...], wd_ref[...],
                          (((1,), (0,)), ((), ())),
                          preferred_element_type=jnp.float32))


def _tc_node_body(fi_ref, fo_ref, wa_ref, wb_ref, b_ref, out_ref):
    acc = lax.dot_general(fi_ref[...], wa_ref[...],
                          (((1,), (0,)), ((), ())),
                          preferred_element_type=jnp.float32)
    acc += lax.dot_general(fo_ref[...], wb_ref[...],
                           (((1,), (0,)), ((), ())),
                           preferred_element_type=jnp.float32)
    out_ref[...] = jnp.maximum(acc + b_ref[...], 0.0)


def _sc_body(n_nodes, epad, d, y2_ref, ea_ref, row_ref, col_ref, out_ref,
             acc, rwb0, rwb1, clb0, clb1, igb0, igb1, irb0, irb1, eab0, eab1,
             yb0, yb1, sr0, sr1, sc0, sc1, se0, se1, sy0, sy1):
    c = lax.axis_index("c")
    s = lax.axis_index("s")
    nacc = n_nodes + TRASH
    zstride = (-(-nacc // NS) + 7) // 8 * 8
    zbase = jnp.minimum(s * zstride, nacc - zstride)
    wstride = (-(-n_nodes // NS) + 7) // 8 * 8
    wbase = jnp.minimum(s * wstride, n_nodes - wstride)
    edges_per_tile = epad // NS
    nsteps = edges_per_tile // CHUNK
    assert nsteps % 2 == 0
    nvec = d // 16

    rwb = (rwb0, rwb1)
    clb = (clb0, clb1)
    igb = (igb0, igb1)
    irb = (irb0, irb1)
    eab = (eab0, eab1)
    yb = (yb0, yb1)
    sr = (sr0, sr1)
    sc_ = (sc0, sc1)
    se = (se0, se1)
    sy = (sy0, sy1)

    ebase = s * edges_per_tile
    sdir = 2 * c - 1

    def issue_lin(t, p):
        pltpu.async_copy(row_ref.at[pl.ds(ebase + t * CHUNK, CHUNK)],
                         rwb[p], sr[p])
        pltpu.async_copy(col_ref.at[pl.ds(ebase + t * CHUNK, CHUNK)],
                         clb[p], sc_[p])
        pltpu.async_copy(ea_ref.at[pl.ds(ebase + t * CHUNK, CHUNK)],
                         eab[p], se[p])

    def gather_start(t, p):
        pltpu.make_async_copy(row_ref.at[pl.ds(ebase, CHUNK)], rwb[p],
                              sr[p]).wait()
        pltpu.make_async_copy(col_ref.at[pl.ds(ebase, CHUNK)], clb[p],
                              sc_[p]).wait()
        tro = t * CHUNK
        for k in range(CHUNK // 16):
            sl = pl.ds(k * 16, 16)
            r = rwb[p][sl]
            cc = clb[p][sl]
            m_out = cc > r
            igb[p][sl] = jnp.where(m_out, cc + n_nodes, cc)
            live = (cc - r) * sdir > 0
            trash = n_nodes + ((tro + k * 16 +
                                lax.iota(jnp.int32, 16)) & (TRASH - 1))
            irb[p][sl] = jnp.where(live, r, trash)
        pltpu.async_copy(y2_ref.at[igb[p]], yb[p], sy[p])

    def compute(p):
        pltpu.make_async_copy(ea_ref.at[pl.ds(ebase, CHUNK)], eab[p],
                              se[p]).wait()
        pltpu.make_async_copy(y2_ref.at[igb[p]], yb[p], sy[p]).wait()

        def crow(r, carry):
            for k in range(nvec):
                sl = pl.ds(k * 16, 16)
                eab[p][r, sl] = jnp.maximum(eab[p][r, sl] + yb[p][r, sl],
                                            0.0)
            return carry
        lax.fori_loop(0, CHUNK, crow, 0)

    def scat(p):
        pltpu.sync_copy(eab[p], acc.at[irb[p]], add=True)

    def zrow(r, carry):
        for k in range(nvec):
            eab0[r, pl.ds(k * 16, 16)] = jnp.zeros((16,), jnp.float32)
        return carry
    lax.fori_loop(0, CHUNK, zrow, 0)
    full, rem = divmod(zstride, CHUNK)
    for j in range(full):
        pltpu.sync_copy(eab0, acc.at[pl.ds(zbase + j * CHUNK, CHUNK)])
    if rem:
        pltpu.sync_copy(eab0.at[pl.ds(0, rem)],
                        acc.at[pl.ds(zbase + full * CHUNK, rem)])
    plsc.subcore_barrier()

    issue_lin(0, 0)
    gather_start(0, 0)

    def pair(tt, carry):
        t0 = 2 * tt
        issue_lin(t0 + 1, 1)
        compute(0)
        gather_start(t0 + 1, 1)
        scat(0)

        @pl.when(t0 + 2 < nsteps)
        def _():
            issue_lin(t0 + 2, 0)
        compute(1)

        @pl.when(t0 + 2 < nsteps)
        def _():
            gather_start(t0 + 2, 0)
        scat(1)
        return carry
    lax.fori_loop(0, nsteps // 2, pair, 0)

    plsc.subcore_barrier()
    pltpu.sync_copy(acc.at[pl.ds(wbase, wstride)],
                    out_ref.at[pl.ds(c * n_nodes + wbase, wstride)])


def kernel(x, edge_index, edge_attr, W_in, b_in, W_out, b_out, W_node,
           b_node):
    n, d = x.shape
    e = edge_index.shape[1]
    de = edge_attr.shape[1]
    gran = NS * CHUNK * 2
    assert gran % EBLK == 0 and e % EBLK == 0
    epad = ((e + gran - 1) // gran) * gran
    neblk = epad // EBLK
    nfull = e // EBLK

    padv = jnp.arange(epad - e, dtype=jnp.int32) % n
    rowp = jnp.concatenate([edge_index[0], padv])
    colp = jnp.concatenate([edge_index[1], padv])
    eam = edge_attr * (edge_index[1] > edge_index[0]
                       ).astype(jnp.float32)[:, None]

    wx = jnp.stack([W_in[:d], W_out[:d]])
    b2 = jnp.stack([b_in, b_out]).reshape(2, 1, d)
    nblk_y = 10
    rows_y = n // nblk_y
    y2 = pl.pallas_call(
        _tc_y2_body,
        grid=(2, nblk_y),
        in_specs=[
            pl.BlockSpec((rows_y, d), lambda dd, i: (i, 0)),
            pl.BlockSpec((1, d, d), lambda dd, i: (dd, 0, 0)),
            pl.BlockSpec((1, 1, d), lambda dd, i: (dd, 0, 0)),
        ],
        out_specs=pl.BlockSpec((rows_y, d),
                               lambda dd, i: (dd * nblk_y + i, 0)),
        out_shape=jax.ShapeDtypeStruct((2 * n, d), jnp.float32),
    )(x, wx, b2)

    ea_sel = pl.pallas_call(
        _tc_edge_body,
        grid=(neblk,),
        in_specs=[
            pl.BlockSpec((EBLK, de),
                         lambda i: (jnp.minimum(i, nfull - 1), 0)),
            pl.BlockSpec((EBLK, de),
                         lambda i: (jnp.minimum(i, nfull - 1), 0)),
            pl.BlockSpec((de, d), lambda i: (0, 0)),
            pl.BlockSpec((de, d), lambda i: (0, 0)),
        ],
        out_specs=pl.BlockSpec((EBLK, d), lambda i: (i, 0)),
        out_shape=jax.ShapeDtypeStruct((epad, d), jnp.float32),
    )(edge_attr, eam, W_in[d:], W_out[d:] - W_in[d:])

    mesh = plsc.VectorSubcoreMesh(core_axis_name="c", subcore_axis_name="s",
                                  num_cores=NC, num_subcores=NS)
    flow = pl.kernel(
        functools.partial(_sc_body, n, epad, d),
        out_type=jax.ShapeDtypeStruct((2 * n, d), jnp.float32),
        mesh=mesh,
        scratch_types=(
            [pltpu.MemorySpace.VMEM_SHARED((n + TRASH, d), jnp.float32)]
            + [pltpu.VMEM((CHUNK,), jnp.int32)] * 8
            + [pltpu.VMEM((CHUNK, d), jnp.float32)] * 4
            + [pltpu.SemaphoreType.DMA] * 8
        ),
    )(y2, ea_sel, rowp, colp)

    nblk = 10
    rows_n = n // nblk
    out = pl.pallas_call(
        _tc_node_body,
        grid=(nblk,),
        in_specs=[
            pl.BlockSpec((rows_n, d), lambda i: (i, 0)),
            pl.BlockSpec((rows_n, d), lambda i: (nblk + i, 0)),
            pl.BlockSpec((d, d), lambda i: (0, 0)),
            pl.BlockSpec((d, d), lambda i: (1, 0)),
            pl.BlockSpec((1, d), lambda i: (0, 0)),
        ],
        out_specs=pl.BlockSpec((rows_n, d), lambda i: (i, 0)),
        out_shape=jax.ShapeDtypeStruct((n, d), jnp.float32),
    )(flow, flow, W_node, W_node, b_node.reshape(1, d))
    return out
